# baseline (speedup 1.0000x reference)
import functools

import jax
import jax.numpy as jnp
from jax import lax
from jax.experimental import pallas as pl
from jax.experimental.pallas import tpu as pltpu
from jax.experimental.pallas import tpu_sc as plsc

N = 10000
D = 128
E = 320000

NC = 2
NS = 16
NW = NC * NS
EPW = E // NW
CH = 80
ITERS = EPW // CH
NP = 10240
RPS = NP // NS
ZR = 128

ROW_BLK = 1000


def _dot_t(a, w):
    return lax.dot_general(a.astype(jnp.bfloat16), w.astype(jnp.bfloat16),
                           (((1,), (1,)), ((), ())),
                           preferred_element_type=jnp.float32)


def _sigmoid(v):
    return 1.0 / (1.0 + jnp.exp(-v))


def _mnet_body(x_ref, win_ref, bin_ref, wh_ref, bh_ref, wout_ref, o_ref):
    h = jax.nn.relu(_dot_t(x_ref[...], win_ref[...]) + bin_ref[...])
    h = h + jax.nn.relu(_dot_t(h, wh_ref[0]) + bh_ref[0])
    h = h + jax.nn.relu(_dot_t(h, wh_ref[1]) + bh_ref[1])
    o_ref[...] = _dot_t(h, wout_ref[...])


def _run_mnet(x, win, b_in, wh, bh, wout):
    grid = (N // ROW_BLK,)
    full = lambda s: pl.BlockSpec(s, lambda i: (0,) * len(s))
    return pl.pallas_call(
        _mnet_body,
        grid=grid,
        in_specs=[
            pl.BlockSpec((ROW_BLK, D), lambda i: (i, 0)),
            full((D, D)),
            full((1, D)),
            full((2, D, D)),
            full((2, 1, D)),
            full((D, D)),
        ],
        out_specs=pl.BlockSpec((ROW_BLK, D), lambda i: (i, 0)),
        out_shape=jax.ShapeDtypeStruct((N, D), jnp.float32),
    )(x, win.astype(jnp.bfloat16), b_in.reshape(1, D),
      wh.astype(jnp.bfloat16), bh.reshape(2, 1, D),
      wout.astype(jnp.bfloat16))


def _seg_body(msg_hbm, edge_hbm, out_hbm,
              src_v, d0_v, d1_v, r0_v, r1_v, theta_s,
              si0, si1, sg0, sg1):
    cid = lax.axis_index("c")
    sid = lax.axis_index("s")
    wid = cid * NS + sid
    ebase = wid * EPW
    dbase = E + ebase

    pltpu.async_copy(edge_hbm.at[pl.ds(ebase, EPW)], src_v, si0)

    zv = jnp.zeros((16,), jnp.float32)
    def zrow(i, c):
        for j in range(D // 16):
            r0_v[i, pl.ds(j * 16, 16)] = zv
        return c
    lax.fori_loop(0, CH, zrow, 0)
    for k in range(RPS // CH):
        pltpu.sync_copy(r0_v, theta_s.at[pl.ds(sid * RPS + k * CH, CH)])
    pltpu.make_async_copy(edge_hbm.at[pl.ds(ebase, EPW)], src_v, si0).wait()
    plsc.subcore_barrier()

    def istart(i, dbuf, sem):
        pltpu.async_copy(edge_hbm.at[pl.ds(dbase + i * CH, CH)], dbuf, sem)

    def iwait(i, dbuf, sem):
        pltpu.make_async_copy(edge_hbm.at[pl.ds(dbase + i * CH, CH)],
                              dbuf, sem).wait()

    def gstart(i, buf, sem):
        pltpu.async_copy(msg_hbm.at[src_v.at[pl.ds(i * CH, CH)]], buf, sem)

    def gwait(i, buf, sem):
        pltpu.make_async_copy(msg_hbm.at[src_v.at[pl.ds(i * CH, CH)]],
                              buf, sem).wait()

    def scat(buf, dbuf):
        pltpu.sync_copy(buf, theta_s.at[dbuf], add=True)

    istart(0, d0_v, si0)
    gstart(0, r0_v, sg0)
    istart(1, d1_v, si1)
    gstart(1, r1_v, sg1)

    def pair(k, c):
        b = 2 * k
        gwait(b, r0_v, sg0)
        iwait(b, d0_v, si0)
        scat(r0_v, d0_v)
        istart(b + 2, d0_v, si0)
        gstart(b + 2, r0_v, sg0)
        gwait(b + 1, r1_v, sg1)
        iwait(b + 1, d1_v, si1)
        scat(r1_v, d1_v)
        @pl.when(b + 3 < ITERS)
        def _():
            istart(b + 3, d1_v, si1)
            gstart(b + 3, r1_v, sg1)
        return c
    lax.fori_loop(0, (ITERS - 1) // 2, pair, 0)
    gwait(ITERS - 1, r0_v, sg0)
    iwait(ITERS - 1, d0_v, si0)
    scat(r0_v, d0_v)
    plsc.subcore_barrier()

    pltpu.sync_copy(theta_s.at[pl.ds(sid * RPS, RPS)],
                    out_hbm.at[cid, pl.ds(sid * RPS, RPS)])


_seg_kernel = functools.partial(
    pl.kernel,
    mesh=plsc.VectorSubcoreMesh(core_axis_name="c", subcore_axis_name="s"),
    out_type=jax.ShapeDtypeStruct((NC, NP, D), jnp.float32),
    scratch_types=[
        pltpu.VMEM((EPW,), jnp.int32),
        pltpu.VMEM((CH,), jnp.int32),
        pltpu.VMEM((CH,), jnp.int32),
        pltpu.VMEM((CH, D), jnp.float32),
        pltpu.VMEM((CH, D), jnp.float32),
        pltpu.VMEM_SHARED((NP, D), jnp.float32),
        pltpu.SemaphoreType.DMA,
        pltpu.SemaphoreType.DMA,
        pltpu.SemaphoreType.DMA,
        pltpu.SemaphoreType.DMA,
    ],
)(_seg_body)


def _update_body(th_ref,
                 swin_ref, sbin_ref, swh_ref, sbh_ref, swout_ref,
                 rwin_ref, rbin_ref, rwh_ref, rbh_ref, rwout_ref,
                 ewin_ref, ebin_ref, ewh_ref, ebh_ref, ewout_ref,
                 o_ref):
    th = th_ref[0] + th_ref[1]

    s = _sigmoid(_dot_t(th, swin_ref[...]) + sbin_ref[...])
    s = _sigmoid(_dot_t(s, swh_ref[0]) + sbh_ref[0])
    s = _sigmoid(_dot_t(s, swh_ref[1]) + sbh_ref[1])
    s_out = _dot_t(s, swout_ref[...])[:, 0:1]

    r = jax.nn.relu(_dot_t(th, rwin_ref[...]) + rbin_ref[...])
    r = r + jax.nn.relu(_dot_t(r, rwh_ref[0]) + rbh_ref[0])
    r = r + jax.nn.relu(_dot_t(r, rwh_ref[1]) + rbh_ref[1])
    r_out = _dot_t(r, rwout_ref[...])

    e = jax.nn.relu(_dot_t(th, ewin_ref[...]) + ebin_ref[...])
    e = jax.nn.relu(_dot_t(e, ewh_ref[0]) + ebh_ref[0])
    e = jax.nn.relu(_dot_t(e, ewh_ref[1]) + ebh_ref[1])
    e_out = _dot_t(e, ewout_ref[...])

    o_ref[...] = s_out * (r_out + e_out * th)


def _run_update(theta2,
                s_win, s_bin, s_wh, s_bh, s_wout,
                r_win, r_bin, r_wh, r_bh, r_wout,
                e_win, e_bin, e_wh, e_bh, e_wout):
    grid = (N // ROW_BLK,)
    full = lambda s: pl.BlockSpec(s, lambda i: (0,) * len(s))
    net = lambda wout_shape: [
        full((D, D)), full((1, D)), full((2, D, D)), full((2, 1, D)),
        full(wout_shape),
    ]
    _call = pl.pallas_call(
        _update_body,
        grid=grid,
        in_specs=[pl.BlockSpec((NC, ROW_BLK, D), lambda i: (0, i, 0))]
        + net((8, D)) + net((D, D)) + net((D, D)),
        out_specs=pl.BlockSpec((ROW_BLK, D), lambda i: (i, 0)),
        out_shape=jax.ShapeDtypeStruct((N, D), jnp.float32),
    )
    bf = lambda w: w.astype(jnp.bfloat16)
    return _call(theta2,
                 bf(s_win), s_bin.reshape(1, D), bf(s_wh),
                 s_bh.reshape(2, 1, D), bf(jnp.pad(s_wout, ((0, 7), (0, 0)))),
                 bf(r_win), r_bin.reshape(1, D), bf(r_wh),
                 r_bh.reshape(2, 1, D), bf(r_wout),
                 bf(e_win), e_bin.reshape(1, D), bf(e_wh),
                 e_bh.reshape(2, 1, D), bf(e_wout))


def kernel(x, edge_index,
           M_Win, M_bin, M_Wh, M_bh, M_Wout,
           S_Win, S_bin, S_Wh, S_bh, S_Wout,
           R_Win, R_bin, R_Wh, R_bh, R_Wout,
           E_Win, E_bin, E_Wh, E_bh, E_Wout):
    msg = _run_mnet(x, M_Win, M_bin, M_Wh, M_bh, M_Wout)
    theta2 = _seg_kernel(msg, edge_index.reshape(-1))
    return _run_update(theta2,
                       S_Win, S_bin, S_Wh, S_bh, S_Wout,
                       R_Win, R_bin, R_Wh, R_bh, R_Wout,
                       E_Win, E_bin, E_Wh, E_bh, E_Wout)


# speedup vs baseline: 1.0268x; 1.0268x over previous
import functools

import jax
import jax.numpy as jnp
from jax import lax
from jax.experimental import pallas as pl
from jax.experimental.pallas import tpu as pltpu
from jax.experimental.pallas import tpu_sc as plsc

N = 10000
D = 128
E = 320000

NC = 2
NS = 16
NW = NC * NS
EPW = E // NW
CH = 80
ITERS = EPW // CH
NP = 10240
RPS = NP // NS
ZR = 128

ROW_BLK = 1000


def _dot_t(a, w):
    return lax.dot_general(a.astype(jnp.bfloat16), w.astype(jnp.bfloat16),
                           (((1,), (1,)), ((), ())),
                           preferred_element_type=jnp.float32)


def _sigmoid(v):
    return 1.0 / (1.0 + jnp.exp(-v))


def _mnet_body(x_ref, win_ref, bin_ref, wh_ref, bh_ref, wout_ref, o_ref):
    h = jax.nn.relu(_dot_t(x_ref[...], win_ref[...]) + bin_ref[...])
    h = h + jax.nn.relu(_dot_t(h, wh_ref[0]) + bh_ref[0])
    h = h + jax.nn.relu(_dot_t(h, wh_ref[1]) + bh_ref[1])
    o_ref[...] = _dot_t(h, wout_ref[...])


def _run_mnet(x, win, b_in, wh, bh, wout):
    grid = (N // ROW_BLK,)
    full = lambda s: pl.BlockSpec(s, lambda i: (0,) * len(s))
    return pl.pallas_call(
        _mnet_body,
        grid=grid,
        in_specs=[
            pl.BlockSpec((ROW_BLK, D), lambda i: (i, 0)),
            full((D, D)),
            full((1, D)),
            full((2, D, D)),
            full((2, 1, D)),
            full((D, D)),
        ],
        out_specs=pl.BlockSpec((ROW_BLK, D), lambda i: (i, 0)),
        out_shape=jax.ShapeDtypeStruct((N, D), jnp.float32),
    )(x, win, b_in.reshape(1, D), wh, bh.reshape(2, 1, D), wout)


def _seg_body(msg_hbm, edge_hbm, out_hbm,
              src_v, d0_v, d1_v, r0_v, r1_v, theta_s,
              si0, si1, sg0, sg1):
    cid = lax.axis_index("c")
    sid = lax.axis_index("s")
    wid = cid * NS + sid
    ebase = wid * EPW
    dbase = E + ebase

    pltpu.async_copy(edge_hbm.at[pl.ds(ebase, EPW)], src_v, si0)

    zv = jnp.zeros((16,), jnp.float32)
    def zrow(i, c):
        for j in range(D // 16):
            r0_v[i, pl.ds(j * 16, 16)] = zv
        return c
    lax.fori_loop(0, CH, zrow, 0)
    for k in range(RPS // CH):
        pltpu.sync_copy(r0_v, theta_s.at[pl.ds(sid * RPS + k * CH, CH)])
    pltpu.make_async_copy(edge_hbm.at[pl.ds(ebase, EPW)], src_v, si0).wait()
    plsc.subcore_barrier()

    def istart(i, dbuf, sem):
        pltpu.async_copy(edge_hbm.at[pl.ds(dbase + i * CH, CH)], dbuf, sem)

    def iwait(i, dbuf, sem):
        pltpu.make_async_copy(edge_hbm.at[pl.ds(dbase + i * CH, CH)],
                              dbuf, sem).wait()

    def gstart(i, buf, sem):
        pltpu.async_copy(msg_hbm.at[src_v.at[pl.ds(i * CH, CH)]], buf, sem)

    def gwait(i, buf, sem):
        pltpu.make_async_copy(msg_hbm.at[src_v.at[pl.ds(i * CH, CH)]],
                              buf, sem).wait()

    def scat(buf, dbuf):
        pltpu.sync_copy(buf, theta_s.at[dbuf], add=True)

    istart(0, d0_v, si0)
    gstart(0, r0_v, sg0)
    istart(1, d1_v, si1)
    gstart(1, r1_v, sg1)

    def pair(k, c):
        b = 2 * k
        gwait(b, r0_v, sg0)
        iwait(b, d0_v, si0)
        scat(r0_v, d0_v)
        istart(b + 2, d0_v, si0)
        gstart(b + 2, r0_v, sg0)
        gwait(b + 1, r1_v, sg1)
        iwait(b + 1, d1_v, si1)
        scat(r1_v, d1_v)
        @pl.when(b + 3 < ITERS)
        def _():
            istart(b + 3, d1_v, si1)
            gstart(b + 3, r1_v, sg1)
        return c
    lax.fori_loop(0, (ITERS - 1) // 2, pair, 0)
    gwait(ITERS - 1, r0_v, sg0)
    iwait(ITERS - 1, d0_v, si0)
    scat(r0_v, d0_v)
    plsc.subcore_barrier()

    pltpu.sync_copy(theta_s.at[pl.ds(sid * RPS, RPS)],
                    out_hbm.at[cid, pl.ds(sid * RPS, RPS)])


_seg_kernel = functools.partial(
    pl.kernel,
    mesh=plsc.VectorSubcoreMesh(core_axis_name="c", subcore_axis_name="s"),
    out_type=jax.ShapeDtypeStruct((NC, NP, D), jnp.float32),
    scratch_types=[
        pltpu.VMEM((EPW,), jnp.int32),
        pltpu.VMEM((CH,), jnp.int32),
        pltpu.VMEM((CH,), jnp.int32),
        pltpu.VMEM((CH, D), jnp.float32),
        pltpu.VMEM((CH, D), jnp.float32),
        pltpu.VMEM_SHARED((NP, D), jnp.float32),
        pltpu.SemaphoreType.DMA,
        pltpu.SemaphoreType.DMA,
        pltpu.SemaphoreType.DMA,
        pltpu.SemaphoreType.DMA,
    ],
)(_seg_body)


def _update_body(th_ref,
                 swin_ref, sbin_ref, swh_ref, sbh_ref, swout_ref,
                 rwin_ref, rbin_ref, rwh_ref, rbh_ref, rwout_ref,
                 ewin_ref, ebin_ref, ewh_ref, ebh_ref, ewout_ref,
                 o_ref):
    th = th_ref[0] + th_ref[1]

    s = _sigmoid(_dot_t(th, swin_ref[...]) + sbin_ref[...])
    s = _sigmoid(_dot_t(s, swh_ref[0]) + sbh_ref[0])
    s = _sigmoid(_dot_t(s, swh_ref[1]) + sbh_ref[1])
    s_out = _dot_t(s, swout_ref[...])[:, 0:1]

    r = jax.nn.relu(_dot_t(th, rwin_ref[...]) + rbin_ref[...])
    r = r + jax.nn.relu(_dot_t(r, rwh_ref[0]) + rbh_ref[0])
    r = r + jax.nn.relu(_dot_t(r, rwh_ref[1]) + rbh_ref[1])
    r_out = _dot_t(r, rwout_ref[...])

    e = jax.nn.relu(_dot_t(th, ewin_ref[...]) + ebin_ref[...])
    e = jax.nn.relu(_dot_t(e, ewh_ref[0]) + ebh_ref[0])
    e = jax.nn.relu(_dot_t(e, ewh_ref[1]) + ebh_ref[1])
    e_out = _dot_t(e, ewout_ref[...])

    o_ref[...] = s_out * (r_out + e_out * th)


def _run_update(theta2,
                s_win, s_bin, s_wh, s_bh, s_wout,
                r_win, r_bin, r_wh, r_bh, r_wout,
                e_win, e_bin, e_wh, e_bh, e_wout):
    grid = (N // ROW_BLK,)
    full = lambda s: pl.BlockSpec(s, lambda i: (0,) * len(s))
    net = lambda wout_shape: [
        full((D, D)), full((1, D)), full((2, D, D)), full((2, 1, D)),
        full(wout_shape),
    ]
    _call = pl.pallas_call(
        _update_body,
        grid=grid,
        in_specs=[pl.BlockSpec((NC, ROW_BLK, D), lambda i: (0, i, 0))]
        + net((8, D)) + net((D, D)) + net((D, D)),
        out_specs=pl.BlockSpec((ROW_BLK, D), lambda i: (i, 0)),
        out_shape=jax.ShapeDtypeStruct((N, D), jnp.float32),
    )
    return _call(theta2,
                 s_win, s_bin.reshape(1, D), s_wh, s_bh.reshape(2, 1, D),
                 jnp.pad(s_wout, ((0, 7), (0, 0))),
                 r_win, r_bin.reshape(1, D), r_wh, r_bh.reshape(2, 1, D), r_wout,
                 e_win, e_bin.reshape(1, D), e_wh, e_bh.reshape(2, 1, D), e_wout)


def kernel(x, edge_index,
           M_Win, M_bin, M_Wh, M_bh, M_Wout,
           S_Win, S_bin, S_Wh, S_bh, S_Wout,
           R_Win, R_bin, R_Wh, R_bh, R_Wout,
           E_Win, E_bin, E_Wh, E_bh, E_Wout):
    msg = _run_mnet(x, M_Win, M_bin, M_Wh, M_bh, M_Wout)
    theta2 = _seg_kernel(msg, edge_index.reshape(-1))
    return _run_update(theta2,
                       S_Win, S_bin, S_Wh, S_bh, S_Wout,
                       R_Win, R_bin, R_Wh, R_bh, R_Wout,
                       E_Win, E_bin, E_Wh, E_bh, E_Wout)
